# Initial kernel scaffold; baseline (speedup 1.0000x reference)
import functools

import jax
import jax.numpy as jnp
from jax import lax
from jax.experimental import pallas as pl
from jax.experimental.pallas import tpu as pltpu
from jax.experimental.pallas import tpu_sc as plsc

B, S, D, V = 4096, 50, 128, 1000
TINY = 1e-35

BB = 256
NW = 32
RPW = B // NW
R = 16
CH = RPW // R
SR = S * R
NV = SR // 16


def _tc_body(s2t_ref, tok_ref, mask_ref, enc_ref, dh_ref, wg_ref, bg_ref,
             wc_ref, bc_ref, gen0_ref, g_ref, clp_ref, t_ref):
    h = dh_ref[...]
    gs = jnp.dot(h, wg_ref[...], preferred_element_type=jnp.float32) + bg_ref[...]
    enc = enc_ref[...]
    proj = jnp.tanh(
        jnp.dot(enc.reshape(BB * S, D), wc_ref[...],
                preferred_element_type=jnp.float32) + bc_ref[...])
    cs = jnp.sum(proj.reshape(BB, S, D) * h[:, None, :], axis=2)
    csm = cs + jnp.log(mask_ref[...] + TINY)

    m0 = jnp.maximum(jnp.max(gs, axis=1), jnp.max(csm, axis=1))
    se = (jnp.sum(jnp.exp(gs - m0[:, None]), axis=1)
          + jnp.sum(jnp.exp(csm - m0[:, None]), axis=1))
    logz = m0 + jnp.log(se)
    gen0 = gs - logz[:, None]
    clp = csm - logz[:, None]

    m = jnp.max(clp, axis=1)
    p = jnp.exp(clp - m[:, None])
    s2t = s2t_ref[...]
    tok = tok_ref[...]

    eq_s = s2t[:, :, None] == s2t[:, None, :]
    tsum = jnp.sum(jnp.where(eq_s, p[:, None, :], 0.0), axis=2)
    tval = m[:, None] + jnp.log(tsum)

    ii = lax.broadcasted_iota(jnp.int32, (S, S), 0)
    jj = lax.broadcasted_iota(jnp.int32, (S, S), 1)
    le = (jj <= ii)[None]
    gt = (jj > ii)[None]
    eq_t = tok[:, :, None] == tok[:, None, :]
    presum = jnp.sum(jnp.where(eq_t & le, p[:, None, :], 0.0), axis=2)
    pre = m[:, None] + jnp.log(presum)
    futc = jnp.sum(jnp.where(eq_t & gt, 1.0, 0.0), axis=2)
    add_mask = ((s2t != 0) & (s2t != 1) & (s2t != 2)
                & (s2t != 3)).astype(jnp.float32)
    dup0 = (futc == 0.0).astype(jnp.float32)
    modified = (pre + jnp.log(dup0 + TINY)
                + jnp.log((1.0 - add_mask) + TINY))

    gen0_ref[...] = gen0
    clp_ref[...] = clp
    t_ref[...] = tval
    g_ref[...] = jnp.concatenate([gen0, modified], axis=1)


_tc_call = pl.pallas_call(
    _tc_body,
    grid=(B // BB,),
    in_specs=[
        pl.BlockSpec((BB, S), lambda i: (i, 0)),
        pl.BlockSpec((BB, S), lambda i: (i, 0)),
        pl.BlockSpec((BB, S), lambda i: (i, 0)),
        pl.BlockSpec((BB, S, D), lambda i: (i, 0, 0)),
        pl.BlockSpec((BB, D), lambda i: (i, 0)),
        pl.BlockSpec((D, V), lambda i: (0, 0)),
        pl.BlockSpec((1, V), lambda i: (0, 0)),
        pl.BlockSpec((D, D), lambda i: (0, 0)),
        pl.BlockSpec((1, D), lambda i: (0, 0)),
    ],
    out_specs=[
        pl.BlockSpec((BB, V), lambda i: (i, 0)),
        pl.BlockSpec((BB, V + S), lambda i: (i, 0)),
        pl.BlockSpec((BB, S), lambda i: (i, 0)),
        pl.BlockSpec((BB, S), lambda i: (i, 0)),
    ],
    out_shape=[
        jax.ShapeDtypeStruct((B, V), jnp.float32),
        jax.ShapeDtypeStruct((B, V + S), jnp.float32),
        jax.ShapeDtypeStruct((B, S), jnp.float32),
        jax.ShapeDtypeStruct((B, S), jnp.float32),
    ],
)


@functools.partial(
    pl.kernel,
    out_type=jax.ShapeDtypeStruct((B, V + S), jnp.float32),
    mesh=plsc.VectorSubcoreMesh(core_axis_name="c", subcore_axis_name="s"),
    scratch_types=[
        pltpu.VMEM((R, V + S), jnp.float32),
        pltpu.VMEM((R, V), jnp.float32),
        pltpu.VMEM((SR,), jnp.int32),
        pltpu.VMEM((SR,), jnp.float32),
    ],
)
def _sc_scatter(g_hbm, gen0_hbm, ids_hbm, t_hbm, out_hbm, buf, grow, idsv, tvv):
    cid = lax.axis_index("c")
    sid = lax.axis_index("s")
    base = (sid * 2 + cid) * RPW
    lane = jnp.arange(16, dtype=jnp.int32)

    def chunk(c, carry):
        r0 = pl.multiple_of(base + c * R, R)
        pltpu.sync_copy(g_hbm.at[pl.ds(r0, R)], buf)
        pltpu.sync_copy(gen0_hbm.at[pl.ds(r0, R)], grow)
        pltpu.sync_copy(ids_hbm.at[pl.ds(r0 * S, SR)], idsv)
        pltpu.sync_copy(t_hbm.at[pl.ds(r0 * S, SR)], tvv)
        for k in range(NV):
            ids = idsv[pl.ds(k * 16, 16)]
            tv = tvv[pl.ds(k * 16, 16)]
            row = (lane + k * 16) // S
            g = plsc.load_gather(grow, [row, ids])
            hi = jnp.maximum(g, tv)
            z = jnp.exp(jnp.minimum(g, tv) - hi)
            sf = z / (2.0 + z)
            s2 = sf * sf
            poly = 1.0 / 9.0
            poly = poly * s2 + 1.0 / 7.0
            poly = poly * s2 + 1.0 / 5.0
            poly = poly * s2 + 1.0 / 3.0
            poly = poly * s2 + 1.0
            cv = hi + 2.0 * sf * poly
            amask = (ids != 0) & (ids != 1) & (ids != 2) & (ids != 3)
            plsc.store_scatter(buf, [row, ids], cv, mask=amask)
        pltpu.sync_copy(buf, out_hbm.at[pl.ds(r0, R)])
        return carry

    lax.fori_loop(0, CH, chunk, 0)


def kernel(source_token_ids, source2target_ids, source_mask, encoder_output,
           decoder_hidden, W_gen, b_gen, W_copy, b_copy):
    gen0, g, clp, t = _tc_call(
        source2target_ids, source_token_ids, source_mask, encoder_output,
        decoder_hidden, W_gen, b_gen.reshape(1, V), W_copy,
        b_copy.reshape(1, D))
    final = _sc_scatter(g, gen0,
                        source2target_ids.reshape(B * S).astype(jnp.int32),
                        t.reshape(B * S))
    return final, gen0, clp


# speedup vs baseline: 9.4376x; 9.4376x over previous
import functools

import jax
import jax.numpy as jnp
from jax import lax
from jax.experimental import pallas as pl
from jax.experimental.pallas import tpu as pltpu
from jax.experimental.pallas import tpu_sc as plsc

B, S, D, V = 4096, 50, 128, 1000
VP = 1024
TINY = 1e-35
LOG_TINY = -80.59048461914062

BB = 256
NW = 32
RPW = B // NW
R = 16
CH = RPW // R
SR = S * R
NV = SR // 16


def _tc_body(s2tT_ref, tokT_ref, maskT_ref, enc_ref, dh_ref, wg_ref, bg_ref,
             wc_ref, bc_ref, gen0_ref, g_ref, clpT_ref, tT_ref):
    h = dh_ref[...]
    gs = jnp.dot(h, wg_ref[...], preferred_element_type=jnp.float32) + bg_ref[...]
    enc = enc_ref[...]
    proj = jnp.tanh(
        jnp.dot(enc.reshape(BB * S, D), wc_ref[...],
                preferred_element_type=jnp.float32) + bc_ref[...])
    cs = jnp.sum(proj.reshape(BB, S, D) * h[:, None, :], axis=2)
    csmT = jnp.swapaxes(cs, 0, 1) + jnp.log(maskT_ref[...] + TINY)

    mcsT = jnp.max(csmT, axis=0, keepdims=True)
    scsT = jnp.sum(jnp.exp(csmT - mcsT), axis=0, keepdims=True)
    mcs = jnp.swapaxes(mcsT, 0, 1)
    scs = jnp.swapaxes(scsT, 0, 1)
    m0 = jnp.maximum(jnp.max(gs, axis=1, keepdims=True), mcs)
    se = (jnp.sum(jnp.exp(gs - m0), axis=1, keepdims=True)
          + jnp.exp(mcs - m0) * scs)
    logz = m0 + jnp.log(se)
    gen0 = gs[:, :V] - logz

    logzT = jnp.swapaxes(logz, 0, 1)
    clpT = csmT - logzT
    mT = mcsT - logzT
    pT = jnp.exp(clpT - mT)
    s2tT = s2tT_ref[...]
    tokT = tokT_ref[...]

    eq_s = (s2tT[:, None, :] == s2tT[None, :, :]).astype(jnp.float32)
    tsumT = jnp.sum(eq_s * pT[None, :, :], axis=1)
    tvalT = mT + jnp.log(tsumT)

    ii = lax.broadcasted_iota(jnp.int32, (S, S, 1), 0)
    jj = lax.broadcasted_iota(jnp.int32, (S, S, 1), 1)
    tri = (jj <= ii).astype(jnp.float32)
    eq_t = (tokT[:, None, :] == tokT[None, :, :]).astype(jnp.float32)
    eq_tri = eq_t * tri
    presumT = jnp.sum(eq_tri * pT[None, :, :], axis=1)
    preT = mT + jnp.log(presumT)
    futcT = jnp.sum(eq_t, axis=1) - jnp.sum(eq_tri, axis=1)
    is_addT = (s2tT != 0) & (s2tT != 1) & (s2tT != 2) & (s2tT != 3)
    modifiedT = (preT + jnp.where(futcT > 0.0, LOG_TINY, 0.0)
                 + jnp.where(is_addT, LOG_TINY, 0.0))

    gen0_ref[...] = gen0
    clpT_ref[...] = clpT
    tT_ref[...] = tvalT
    g_ref[:, :V] = gen0
    g_ref[:, V:] = jnp.swapaxes(modifiedT, 0, 1)


_tc_call = pl.pallas_call(
    _tc_body,
    grid=(B // BB,),
    in_specs=[
        pl.BlockSpec((S, BB), lambda i: (0, i)),
        pl.BlockSpec((S, BB), lambda i: (0, i)),
        pl.BlockSpec((S, BB), lambda i: (0, i)),
        pl.BlockSpec((BB, S, D), lambda i: (i, 0, 0)),
        pl.BlockSpec((BB, D), lambda i: (i, 0)),
        pl.BlockSpec((D, VP), lambda i: (0, 0)),
        pl.BlockSpec((1, VP), lambda i: (0, 0)),
        pl.BlockSpec((D, D), lambda i: (0, 0)),
        pl.BlockSpec((1, D), lambda i: (0, 0)),
    ],
    out_specs=[
        pl.BlockSpec((BB, V), lambda i: (i, 0)),
        pl.BlockSpec((BB, V + S), lambda i: (i, 0)),
        pl.BlockSpec((S, BB), lambda i: (0, i)),
        pl.BlockSpec((S, BB), lambda i: (0, i)),
    ],
    out_shape=[
        jax.ShapeDtypeStruct((B, V), jnp.float32),
        jax.ShapeDtypeStruct((B, V + S), jnp.float32),
        jax.ShapeDtypeStruct((S, B), jnp.float32),
        jax.ShapeDtypeStruct((S, B), jnp.float32),
    ],
)


@functools.cache
def _sc_scatter_fn():
  return functools.partial(
      pl.kernel,
      out_type=jax.ShapeDtypeStruct((B, V + S), jnp.float32),
      mesh=plsc.VectorSubcoreMesh(core_axis_name="c", subcore_axis_name="s"),
      compiler_params=pltpu.CompilerParams(needs_layout_passes=False),
      scratch_types=[
          pltpu.VMEM((R, V + S), jnp.float32),
          pltpu.VMEM((R, V + S), jnp.float32),
          pltpu.VMEM((SR,), jnp.int32),
          pltpu.VMEM((SR,), jnp.int32),
          pltpu.VMEM((SR,), jnp.float32),
          pltpu.VMEM((SR,), jnp.float32),
          pltpu.VMEM((SR,), jnp.float32),
          pltpu.SemaphoreType.DMA,
          pltpu.SemaphoreType.DMA,
          pltpu.SemaphoreType.DMA,
          pltpu.SemaphoreType.DMA,
      ],
  )(_sc_body)


def _sc_body(g_hbm, ids_hbm, t_hbm, out_hbm, bufA, bufB, idsA, idsB, tA, tB,
             gbuf, inA, inB, outA, outB):
    cid = lax.axis_index("c")
    sid = lax.axis_index("s")
    base = (sid * 2 + cid) * RPW
    lane = jnp.arange(16, dtype=jnp.int32)
    slots = ((bufA, idsA, tA, inA, outA), (bufB, idsB, tB, inB, outB))

    def start_in(c, slot):
        buf, idsv, tvv, insem, _ = slots[slot]
        r0 = pl.multiple_of(base + c * R, R)
        pltpu.make_async_copy(g_hbm.at[pl.ds(r0, R)], buf, insem).start()
        pltpu.make_async_copy(ids_hbm.at[pl.ds(r0 * S, SR)], idsv, insem).start()
        pltpu.make_async_copy(t_hbm.at[pl.ds(r0 * S, SR)], tvv, insem).start()

    def wait_in(slot):
        buf, idsv, tvv, insem, _ = slots[slot]
        pltpu.make_async_copy(g_hbm.at[pl.ds(0, R)], buf, insem).wait()
        pltpu.make_async_copy(ids_hbm.at[pl.ds(0, SR)], idsv, insem).wait()
        pltpu.make_async_copy(t_hbm.at[pl.ds(0, SR)], tvv, insem).wait()

    def start_out(c, slot):
        buf, _, _, _, outsem = slots[slot]
        r0 = pl.multiple_of(base + c * R, R)
        pltpu.make_async_copy(buf, out_hbm.at[pl.ds(r0, R)], outsem).start()

    def wait_out(slot):
        buf, _, _, _, outsem = slots[slot]
        pltpu.make_async_copy(buf, out_hbm.at[pl.ds(0, R)], outsem).wait()

    def compute(slot):
        buf, idsv, tvv, _, _ = slots[slot]
        for k in range(NV):
            ids = idsv[pl.ds(k * 16, 16)]
            row = (lane + k * 16) // S
            gbuf[pl.ds(k * 16, 16)] = plsc.load_gather(buf, [row, ids])
        for k in range(NV):
            ids = idsv[pl.ds(k * 16, 16)]
            tv = tvv[pl.ds(k * 16, 16)]
            g = gbuf[pl.ds(k * 16, 16)]
            row = (lane + k * 16) // S
            hi = jnp.maximum(g, tv)
            z = jnp.exp(jnp.minimum(g, tv) - hi)
            sf = z / (2.0 + z)
            s2 = sf * sf
            poly = 1.0 / 9.0
            poly = poly * s2 + 1.0 / 7.0
            poly = poly * s2 + 1.0 / 5.0
            poly = poly * s2 + 1.0 / 3.0
            poly = poly * s2 + 1.0
            cv = hi + 2.0 * sf * poly
            amask = (ids != 0) & (ids != 1) & (ids != 2) & (ids != 3)
            plsc.store_scatter(buf, [row, ids], cv, mask=amask)

    start_in(0, 0)
    start_in(1, 1)

    def pair(c2, carry):
        c = c2 * 2
        wait_in(0)
        compute(0)
        start_out(c, 0)
        wait_in(1)
        compute(1)
        start_out(c + 1, 1)
        wait_out(0)
        start_in(c + 2, 0)
        wait_out(1)
        start_in(c + 3, 1)
        return carry

    lax.fori_loop(0, CH // 2 - 1, pair, 0)
    c = CH - 2
    wait_in(0)
    compute(0)
    start_out(c, 0)
    wait_in(1)
    compute(1)
    start_out(c + 1, 1)
    wait_out(0)
    wait_out(1)


def kernel(source_token_ids, source2target_ids, source_mask, encoder_output,
           decoder_hidden, W_gen, b_gen, W_copy, b_copy):
    wg_p = jnp.pad(W_gen, ((0, 0), (0, VP - V)))
    bg_p = jnp.pad(b_gen.reshape(1, V), ((0, 0), (0, VP - V)),
                   constant_values=-1e30)
    gen0, g, clpT, tT = _tc_call(
        source2target_ids.T, source_token_ids.T, source_mask.T, encoder_output,
        decoder_hidden, wg_p, bg_p, W_copy, b_copy.reshape(1, D))
    ids_flat = source2target_ids.reshape(B * S)
    t_flat = jnp.swapaxes(tT, 0, 1).reshape(B * S)
    final = _sc_scatter_fn()(g, ids_flat, t_flat)
    return final, gen0, clpT.T
